# Initial kernel scaffold; baseline (speedup 1.0000x reference)
import jax
import jax.numpy as jnp
from jax.experimental import pallas as pl


def kernel(input_ids, table):
    raise NotImplementedError("write your pallas kernel here")


# speedup vs baseline: 1.3061x; 1.3061x over previous
import functools

import jax
import jax.numpy as jnp
from jax import lax
from jax.experimental import pallas as pl
from jax.experimental.pallas import tpu as pltpu
from jax.experimental.pallas import tpu_sc as plsc

VOCAB = 1000000
LATENT = 32
BATCH = 4096
SEQ = 200

NC = 2
NS = 16
NW = NC * NS

B = BATCH * SEQ
CHUNK = 128
N_CHUNKS = B // CHUNK
CPW = N_CHUNKS // NW


def _make_gather():
  mesh = plsc.VectorSubcoreMesh(core_axis_name="c", subcore_axis_name="s")

  @functools.partial(
      pl.kernel,
      out_type=jax.ShapeDtypeStruct((B, LATENT), jnp.float32),
      mesh=mesh,
      compiler_params=pltpu.CompilerParams(use_tc_tiling_on_sc=False),
      scratch_types=[
          pltpu.VMEM((CPW, CHUNK), jnp.int32),
          pltpu.VMEM((CHUNK, LATENT), jnp.float32),
          pltpu.SemaphoreType.DMA,
      ],
  )
  def gather_kernel(table_hbm, idx_hbm, out_hbm, idx_v, rows_v, sem):
    wid = lax.axis_index("s") * NC + lax.axis_index("c")
    chunk0 = wid * CPW
    pltpu.sync_copy(idx_hbm.at[pl.ds(chunk0, CPW)], idx_v)

    def body(j, _):
      pltpu.async_copy(table_hbm.at[idx_v.at[j]], rows_v, sem).wait()
      pltpu.sync_copy(rows_v, out_hbm.at[pl.ds((chunk0 + j) * CHUNK, CHUNK)])
      return _

    lax.fori_loop(0, CPW, body, None)

  return gather_kernel


_gather = _make_gather()


@jax.jit
def kernel(input_ids, table):
  idx = input_ids.reshape(N_CHUNKS, CHUNK).astype(jnp.int32)
  out = _gather(table, idx)
  return out.reshape(BATCH, SEQ, LATENT)


# speedup vs baseline: 1.5007x; 1.1490x over previous
import functools

import jax
import jax.numpy as jnp
from jax import lax
from jax.experimental import pallas as pl
from jax.experimental.pallas import tpu as pltpu
from jax.experimental.pallas import tpu_sc as plsc

VOCAB = 1000000
LATENT = 32
BATCH = 4096
SEQ = 200

NC = 2
NS = 16
NW = NC * NS

B = BATCH * SEQ
CHUNK = 128
N_CHUNKS = B // CHUNK
CPW = N_CHUNKS // NW

G = 5
ROWS_G = G * CHUNK
NGRP = CPW // G
NBUF = 4
NBLK = NGRP // NBUF


def _make_gather():
  mesh = plsc.VectorSubcoreMesh(core_axis_name="c", subcore_axis_name="s")

  @functools.partial(
      pl.kernel,
      out_type=jax.ShapeDtypeStruct((B, LATENT), jnp.float32),
      mesh=mesh,
      compiler_params=pltpu.CompilerParams(use_tc_tiling_on_sc=False),
      scratch_types=[
          pltpu.VMEM((CPW, CHUNK), jnp.int32),
          [pltpu.VMEM((ROWS_G, LATENT), jnp.float32) for _ in range(NBUF)],
          [pltpu.SemaphoreType.DMA for _ in range(NBUF)],
          [pltpu.SemaphoreType.DMA for _ in range(NBUF)],
      ],
  )
  def gather_kernel(table_hbm, idx_hbm, out_hbm, idx_v, bufs, gsems, ssems):
    wid = lax.axis_index("s") * NC + lax.axis_index("c")
    chunk0 = wid * CPW
    row0 = chunk0 * CHUNK
    pltpu.sync_copy(idx_hbm.at[pl.ds(chunk0, CPW)], idx_v)

    def fire(g, b):
      for i in range(G):
        pltpu.async_copy(
            table_hbm.at[idx_v.at[g * G + i]],
            bufs[b].at[pl.ds(i * CHUNK, CHUNK)],
            gsems[b],
        )

    def drain_gathers(b):
      pltpu.make_async_copy(
          out_hbm.at[pl.ds(0, ROWS_G)], bufs[b], gsems[b]
      ).wait()

    def store(g, b):
      return pltpu.async_copy(
          bufs[b], out_hbm.at[pl.ds(row0 + g * ROWS_G, ROWS_G)], ssems[b]
      )

    for b in range(NBUF):
      fire(b, b)

    def block(bb, _):
      for b in range(NBUF):
        g = bb * NBUF + b
        drain_gathers(b)
        store(g, b).wait()
        fire(g + NBUF, b)
      return _

    lax.fori_loop(0, NBLK - 1, block, None)

    for b in range(NBUF):
      g = (NBLK - 1) * NBUF + b
      drain_gathers(b)
      store(g, b).wait()

  return gather_kernel


_gather = _make_gather()


@jax.jit
def kernel(input_ids, table):
  idx = input_ids.reshape(N_CHUNKS, CHUNK).astype(jnp.int32)
  out = _gather(table, idx)
  return out.reshape(BATCH, SEQ, LATENT)


# speedup vs baseline: 1.5015x; 1.0005x over previous
import functools

import jax
import jax.numpy as jnp
from jax import lax
from jax.experimental import layout as jlayout
from jax.experimental import pallas as pl
from jax.experimental.pallas import tpu as pltpu
from jax.experimental.pallas import tpu_sc as plsc

VOCAB = 1000000
LATENT = 32
BATCH = 4096
SEQ = 200

NC = 2
NS = 16
NW = NC * NS

B = BATCH * SEQ
CHUNK = 128
N_CHUNKS = B // CHUNK
CPW = N_CHUNKS // NW

G = 5
ROWS_G = G * CHUNK
NGRP = CPW // G
NBUF = 4
NBLK = NGRP // NBUF


def _make_gather():
  mesh = plsc.VectorSubcoreMesh(core_axis_name="c", subcore_axis_name="s")

  @functools.partial(
      pl.kernel,
      out_type=jax.ShapeDtypeStruct((B, LATENT), jnp.float32),
      mesh=mesh,
      compiler_params=pltpu.CompilerParams(use_tc_tiling_on_sc=False),
      scratch_types=[
          pltpu.VMEM((CPW, CHUNK), jnp.int32),
          [pltpu.VMEM((ROWS_G, LATENT), jnp.float32) for _ in range(NBUF)],
          [pltpu.SemaphoreType.DMA for _ in range(NBUF)],
          [pltpu.SemaphoreType.DMA for _ in range(NBUF)],
      ],
  )
  def gather_kernel(table_hbm, idx_hbm, out_hbm, idx_v, bufs, gsems, ssems):
    wid = lax.axis_index("s") * NC + lax.axis_index("c")
    chunk0 = wid * CPW
    row0 = chunk0 * CHUNK
    pltpu.sync_copy(idx_hbm.at[pl.ds(chunk0, CPW)], idx_v)

    def fire(g, b):
      for i in range(G):
        pltpu.async_copy(
            table_hbm.at[idx_v.at[g * G + i]],
            bufs[b].at[pl.ds(i * CHUNK, CHUNK)],
            gsems[b],
        )

    def drain_gathers(b):
      pltpu.make_async_copy(
          out_hbm.at[pl.ds(0, ROWS_G)], bufs[b], gsems[b]
      ).wait()

    def store(g, b):
      return pltpu.async_copy(
          bufs[b], out_hbm.at[pl.ds(row0 + g * ROWS_G, ROWS_G)], ssems[b]
      )

    for b in range(NBUF):
      fire(b, b)

    def block(bb, _):
      for b in range(NBUF):
        g = bb * NBUF + b
        drain_gathers(b)
        store(g, b).wait()
        fire(g + NBUF, b)
      return _

    lax.fori_loop(0, NBLK - 1, block, None)

    for b in range(NBUF):
      g = (NBLK - 1) * NBUF + b
      drain_gathers(b)
      store(g, b).wait()

  return gather_kernel


_gather = _make_gather()


def _impl(input_ids, table):
  idx = input_ids.reshape(N_CHUNKS, CHUNK).astype(jnp.int32)
  out = _gather(table, idx)
  return out.reshape(BATCH, SEQ, LATENT)


@functools.lru_cache(maxsize=None)
def _jitted(dev):
  if dev is None:
    return jax.jit(_impl)
  fmt = jlayout.Format(
      jlayout.Layout(major_to_minor=(0, 1, 2)),
      jax.sharding.SingleDeviceSharding(dev),
  )
  return jax.jit(_impl, out_shardings=fmt)


def kernel(input_ids, table):
  dev = None
  try:
    d = next(iter(table.devices()))
    if d.platform.lower() == "tpu":
      dev = d
  except Exception:
    pass
  return _jitted(dev)(input_ids, table)
